# Initial kernel scaffold; baseline (speedup 1.0000x reference)
import jax
import jax.numpy as jnp
from jax.experimental import pallas as pl


def kernel(inputs, token_table, pos_table):
    raise NotImplementedError("write your pallas kernel here")


# speedup vs baseline: 1.2183x; 1.2183x over previous
import functools

import jax
import jax.numpy as jnp
from jax import lax
from jax.experimental import pallas as pl
from jax.experimental.pallas import tpu as pltpu
from jax.experimental.pallas import tpu_sc as plsc

EMBED = 32
LANES = 16
NC, NS = 2, 16
NW = NC * NS


def _sc_embed(flat_idx, token_table, pos_table, n_tok, seq):
    per_w = n_tok // NW
    chunk_rows = 4
    chunk = chunk_rows * seq
    n_chunks = per_w // chunk

    mesh = plsc.VectorSubcoreMesh(core_axis_name="c", subcore_axis_name="s")

    @functools.partial(
        pl.kernel,
        out_type=jax.ShapeDtypeStruct((n_tok, EMBED), jnp.float32),
        mesh=mesh,
        scratch_types=[
            pltpu.VMEM((seq, EMBED), jnp.float32),
            pltpu.VMEM((chunk,), jnp.int32),
            pltpu.VMEM((chunk, EMBED), jnp.float32),
            pltpu.SemaphoreType.DMA,
        ],
        compiler_params=pltpu.CompilerParams(use_tc_tiling_on_sc=False),
    )
    def k(idx_hbm, tok_hbm, pos_hbm, out_hbm, pos_v, idx_v, rows_v, sem):
        wid = lax.axis_index("s") * NC + lax.axis_index("c")
        base = wid * per_w
        pltpu.sync_copy(pos_hbm, pos_v)

        def chunk_body(ci, carry):
            off = base + ci * chunk
            pltpu.sync_copy(idx_hbm.at[pl.ds(off, chunk)], idx_v)
            pltpu.async_copy(tok_hbm.at[idx_v], rows_v, sem).wait()

            def add_body(j, c2):
                s = lax.rem(j, seq)
                for h in range(EMBED // LANES):
                    pv = pos_v[s, pl.ds(h * LANES, LANES)]
                    plsc.addupdate(rows_v.at[j, pl.ds(h * LANES, LANES)], pv)
                return c2

            lax.fori_loop(0, chunk, add_body, 0)
            pltpu.sync_copy(rows_v, out_hbm.at[pl.ds(off, chunk)])
            return carry

        lax.fori_loop(0, n_chunks, chunk_body, 0)

    return k(flat_idx, token_table, pos_table)


def kernel(inputs, token_table, pos_table):
    b, s = inputs.shape
    flat = inputs.reshape(b * s).astype(jnp.int32)
    out = _sc_embed(flat, token_table, pos_table[:s], b * s, s)
    return out.reshape(b, s, EMBED)


# speedup vs baseline: 1.3933x; 1.1437x over previous
import functools

import jax
import jax.numpy as jnp
from jax import lax
from jax.experimental import pallas as pl
from jax.experimental.pallas import tpu as pltpu
from jax.experimental.pallas import tpu_sc as plsc

EMBED = 32
LANES = 16
NC, NS = 2, 16
NW = NC * NS


def _sc_embed(inputs, token_table, pos_table):
    batch, seq = inputs.shape
    rows_per_w = batch // NW
    chunk_rows = 4
    n_chunks = rows_per_w // chunk_rows

    mesh = plsc.VectorSubcoreMesh(core_axis_name="c", subcore_axis_name="s")

    @functools.partial(
        pl.kernel,
        out_type=jax.ShapeDtypeStruct((batch, seq, EMBED), jnp.float32),
        mesh=mesh,
        scratch_types=[
            pltpu.VMEM((seq, EMBED), jnp.float32),
            pltpu.VMEM((chunk_rows, seq), jnp.int32),
            pltpu.VMEM((chunk_rows, seq, EMBED), jnp.float32),
            pltpu.SemaphoreType.DMA,
        ],
        compiler_params=pltpu.CompilerParams(use_tc_tiling_on_sc=False),
    )
    def k(idx_hbm, tok_hbm, pos_hbm, out_hbm, pos_v, idx_v, rows_v, sem):
        wid = lax.axis_index("s") * NC + lax.axis_index("c")
        row0 = wid * rows_per_w
        pltpu.sync_copy(pos_hbm, pos_v)

        def chunk_body(ci, carry):
            r0 = row0 + ci * chunk_rows
            pltpu.sync_copy(idx_hbm.at[pl.ds(r0, chunk_rows)], idx_v)
            copies = [
                pltpu.async_copy(tok_hbm.at[idx_v.at[r]], rows_v.at[r], sem)
                for r in range(chunk_rows)
            ]
            for c in copies:
                c.wait()

            def add_body(s, c2):
                for h in range(EMBED // LANES):
                    pv = pos_v[s, pl.ds(h * LANES, LANES)]
                    for r in range(chunk_rows):
                        plsc.addupdate(rows_v.at[r, s, pl.ds(h * LANES, LANES)], pv)
                return c2

            lax.fori_loop(0, seq, add_body, 0)
            pltpu.sync_copy(rows_v, out_hbm.at[pl.ds(r0, chunk_rows)])
            return carry

        lax.fori_loop(0, n_chunks, chunk_body, 0)

    return k(inputs, token_table, pos_table)


def kernel(inputs, token_table, pos_table):
    b, s = inputs.shape
    return _sc_embed(inputs.astype(jnp.int32), token_table, pos_table[:s])
